# baseline (speedup 1.0000x reference)
import functools

import jax
import jax.numpy as jnp
from jax import lax
from jax.experimental import pallas as pl
from jax.experimental.pallas import tpu as pltpu
from jax.experimental.pallas import tpu_sc as plsc

D = 32
T = 200
NC = 2
NS = 16
NW = NC * NS


V = 1000000
BLK = 128
N_FULL = V // BLK
TAIL = V - N_FULL * BLK
PER_W_BLOCKS = N_FULL // NW
N_EXTRA = N_FULL - PER_W_BLOCKS * NW


def _transpose_kernel():
  mesh = plsc.VectorSubcoreMesh(
      core_axis_name="c", subcore_axis_name="s", num_cores=NC,
      num_subcores=NS)

  @functools.partial(
      pl.kernel,
      out_type=jax.ShapeDtypeStruct((V * D,), jnp.float32),
      mesh=mesh,
      scratch_types=[
          pltpu.VMEM((2, D, BLK), jnp.float32),
          pltpu.VMEM((2, BLK * D), jnp.float32),
          pltpu.SemaphoreType.DMA((2,)),
          pltpu.SemaphoreType.DMA((2,)),
      ],
      compiler_params=pltpu.CompilerParams(needs_layout_passes=False),
  )
  def k(temb_hbm, tail_hbm, out_hbm, in_v, tr_v, lsem, ssem):
    wid = lax.axis_index("s") * NC + lax.axis_index("c")
    row_ids = [lax.iota(jnp.int32, 16), lax.iota(jnp.int32, 16) + 16]

    def load(blk, s):
      pltpu.async_copy(
          temb_hbm.at[:, pl.ds(blk * BLK, BLK)], in_v.at[s], lsem.at[s])

    def drain_load(s):
      pltpu.make_async_copy(
          temb_hbm.at[:, pl.ds(0, BLK)], in_v.at[s], lsem.at[s]).wait()

    def drain_store(s):
      pltpu.make_async_copy(
          tr_v.at[s], out_hbm.at[pl.ds(0, BLK * D)], ssem.at[s]).wait()

    def transpose_rows(s, n_rows):
      def tr_body(rl, carry):
        col = jnp.full((16,), rl, jnp.int32)
        for h in range(2):
          g = plsc.load_gather(in_v.at[s], [row_ids[h], col])
          tr_v[s, pl.ds(rl * D + h * 16, 16)] = g
        return carry

      lax.fori_loop(0, n_rows, tr_body, 0, unroll=4)

    def finish(blk, s, store_pending):
      drain_load(s)
      @pl.when(store_pending)
      def _():
        drain_store(s)
      transpose_rows(s, BLK)
      pltpu.async_copy(
          tr_v.at[s], out_hbm.at[pl.ds(blk * (BLK * D), BLK * D)],
          ssem.at[s])

    first = wid * PER_W_BLOCKS
    load(first, 0)

    def pair_body(p, carry):
      b0 = first + 2 * p
      load(b0 + 1, 1)
      finish(b0, 0, p > 0)
      @pl.when(p < PER_W_BLOCKS // 2 - 1)
      def _():
        load(b0 + 2, 0)
      finish(b0 + 1, 1, p > 0)
      return carry

    lax.fori_loop(0, PER_W_BLOCKS // 2, pair_body, 0)
    drain_store(0)
    drain_store(1)

    @pl.when(wid >= NW - N_EXTRA)
    def _():
      blk = NW * PER_W_BLOCKS + (wid - (NW - N_EXTRA))
      load(blk, 0)
      drain_load(0)
      transpose_rows(0, BLK)
      pltpu.sync_copy(
          tr_v.at[0], out_hbm.at[pl.ds(blk * (BLK * D), BLK * D)])

    @pl.when(wid == NW - 1)
    def _():
      pltpu.sync_copy(tail_hbm, tr_v.at[0, pl.ds(0, TAIL * D)])
      pltpu.sync_copy(
          tr_v.at[0, pl.ds(0, TAIL * D)],
          out_hbm.at[pl.ds(N_FULL * (BLK * D), TAIL * D)])

  return k

ROWS_PER_CHUNK = 4
CHUNK = ROWS_PER_CHUNK * T
NSLOT = 4
GATHER_SLICES = [(j * 128, 128) for j in range(6)] + [(768, 32)]


def _make_kernel(n_total):
  per_w = n_total // NW
  n_chunks = per_w // CHUNK
  n_groups = n_chunks // NSLOT
  mesh = plsc.VectorSubcoreMesh(
      core_axis_name="c", subcore_axis_name="s", num_cores=NC,
      num_subcores=NS)

  @functools.partial(
      pl.kernel,
      out_type=jax.ShapeDtypeStruct((n_total, D), jnp.float32),
      mesh=mesh,
      scratch_types=[
          pltpu.VMEM((T, D), jnp.float32),
          pltpu.VMEM((NSLOT, CHUNK), jnp.int32),
          pltpu.VMEM((NSLOT, CHUNK, D), jnp.float32),
          pltpu.SemaphoreType.DMA((NSLOT,)),
          pltpu.SemaphoreType.DMA((NSLOT,)),
      ],
      compiler_params=pltpu.CompilerParams(use_tc_tiling_on_sc=False),
  )
  def k(idx_hbm, emb_hbm, pos_hbm, out_hbm, pos_v, idx_v, rows_v, gsem,
        ssem):
    wid = lax.axis_index("s") * NC + lax.axis_index("c")
    wbase = wid * per_w
    pltpu.sync_copy(pos_hbm, pos_v)

    def load(g, s):
      base = wbase + g * CHUNK
      pltpu.sync_copy(idx_hbm.at[pl.ds(base, CHUNK)], idx_v.at[s])
      for off, sz in GATHER_SLICES:
        pltpu.async_copy(
            emb_hbm.at[idx_v.at[s, pl.ds(off, sz)]],
            rows_v.at[s, pl.ds(off, sz)], gsem.at[s])

    def drain_gathers(s):
      pltpu.make_async_copy(
          out_hbm.at[pl.ds(0, CHUNK)], rows_v.at[s], gsem.at[s]).wait()

    def drain_store(s):
      pltpu.make_async_copy(
          rows_v.at[s], out_hbm.at[pl.ds(0, CHUNK)], ssem.at[s]).wait()

    def finish(g, s):
      drain_gathers(s)

      def add_body(t, carry):
        for d in range(D // 16):
          p = pos_v[t, pl.ds(d * 16, 16)]
          for r in range(ROWS_PER_CHUNK):
            row = r * T + t
            rows_v[s, row, pl.ds(d * 16, 16)] = (
                rows_v[s, row, pl.ds(d * 16, 16)] + p)
        return carry

      lax.fori_loop(0, T, add_body, 0, unroll=2)
      base = wbase + g * CHUNK
      pltpu.async_copy(rows_v.at[s], out_hbm.at[pl.ds(base, CHUNK)],
                       ssem.at[s])

    load(0, 0)

    def group_body(p, carry):
      g0 = p * NSLOT
      for b in range(NSLOT):
        g = g0 + b
        nxt = g + 1
        s_nxt = (b + 1) % NSLOT
        if b == NSLOT - 1:
          @pl.when(p < n_groups - 1)
          def _():
            drain_store(s_nxt)
            load(nxt, s_nxt)
        else:
          @pl.when(p > 0)
          def _():
            drain_store(s_nxt)
          load(nxt, s_nxt)
        finish(g, b)
      return carry

    lax.fori_loop(0, n_groups, group_body, 0)
    for s in range(NSLOT):
      drain_store(s)

  return k


def kernel(x, embedding, pos_encoding):
  b, t = x.shape
  n_total = b * t
  tail = embedding[V - TAIL:, :].reshape(TAIL * D)
  table = _transpose_kernel()(embedding.T, tail)
  out = _make_kernel(n_total)(
      x.reshape(n_total), table.reshape(V, D), pos_encoding)
  return out.reshape(b, t, D)
